# Initial kernel scaffold; baseline (speedup 1.0000x reference)
import jax
import jax.numpy as jnp
from jax.experimental import pallas as pl


def kernel(x_cell, x_atom, x_bond, x_motif, edge_index_atom, edge_index_bond, edge_index_motif, Wl, bl, Wr, Wproj, bproj, Wout, bout):
    raise NotImplementedError("write your pallas kernel here")


# speedup vs baseline: 4.1631x; 4.1631x over previous
import functools

import jax
import jax.numpy as jnp
from jax import lax
from jax.experimental import pallas as pl
from jax.experimental.pallas import tpu as pltpu
from jax.experimental.pallas import tpu_sc as plsc

H = 128
N = 10000
E = 160000
NP = 10112
ROWS_PER_TILE = NP // 16
CHUNK = 128
GROUP = 16
GROUPS = 5
CHUNKS_PER_TILE = GROUP * GROUPS
EPT = 16 * CHUNKS_PER_TILE * CHUNK
N_PASS_PER_SC = 3


def _relu_table_kernel(x_ref, o_ref):
    p = pl.program_id(0)
    x = x_ref[...]
    o_ref[...] = jnp.where(p >= 3, jnp.maximum(x, 0.0), x)


def _build_table(xpad):
    blk = 632
    return pl.pallas_call(
        _relu_table_kernel,
        grid=(6, NP // blk),
        in_specs=[pl.BlockSpec((1, blk, H), lambda p, i: (lax.rem(p, 3), i, 0))],
        out_specs=pl.BlockSpec((1, blk, H), lambda p, i: (p, i, 0)),
        out_shape=jax.ShapeDtypeStruct((6, NP, H), jnp.float32),
    )(xpad)


def _row_chunks():
    out = []
    k = 0
    while k < ROWS_PER_TILE:
        out.append((k, min(CHUNK, ROWS_PER_TILE - k)))
        k += CHUNK
    return out


def _sc_body(table, srcs, dsts, zfeat, ones,
             out_sums, out_cnt,
             src_v, dstc, fbuf, acc, gsem, ssem):
    cid = lax.axis_index("c")
    sid = lax.axis_index("s")
    row0 = sid * ROWS_PER_TILE

    def zero_acc():
        pltpu.sync_copy(zfeat, fbuf)
        for k, sz in _row_chunks():
            pltpu.sync_copy(fbuf.at[pl.ds(0, sz)],
                            acc.at[pl.ds(row0 + k, sz)])

    def copy_out(dst_hbm):
        for k, sz in _row_chunks():
            pltpu.sync_copy(acc.at[pl.ds(row0 + k, sz)],
                            fbuf.at[pl.ds(0, sz)])
            pltpu.sync_copy(fbuf.at[pl.ds(0, sz)],
                            dst_hbm.at[pl.ds(row0 + k, sz)])

    for p_i in range(N_PASS_PER_SC):
        p = cid * N_PASS_PER_SC + p_i
        zero_acc()
        plsc.subcore_barrier()

        for g in range(GROUPS):
            pltpu.sync_copy(srcs.at[p, sid, pl.ds(g * GROUP, GROUP)], src_v)

            def _chunk(j, carry):
                pltpu.sync_copy(dsts.at[p_i, sid, g * GROUP + j], dstc)
                pltpu.async_copy(table.at[src_v.at[j]], fbuf, gsem).wait()
                pltpu.async_copy(fbuf, acc.at[dstc], ssem, add=True).wait()
                return carry

            lax.fori_loop(0, GROUP, _chunk, 0)
        plsc.subcore_barrier()
        copy_out(out_sums.at[p])
        plsc.subcore_barrier()

    def count_pass(r):
        zero_acc()
        plsc.subcore_barrier()
        pltpu.sync_copy(ones, fbuf)

        def _chunk(jj, carry):
            pltpu.sync_copy(
                dsts.at[r, sid, jj], dstc)
            pltpu.async_copy(fbuf, acc.at[dstc], ssem, add=True).wait()
            return carry

        lax.fori_loop(0, GROUPS * GROUP, _chunk, 0)
        plsc.subcore_barrier()
        copy_out(out_cnt.at[r])
        plsc.subcore_barrier()

    @pl.when(cid == 0)
    def _():
        count_pass(0)
        count_pass(1)

    @pl.when(cid == 1)
    def _():
        count_pass(2)


def _sc_aggregate(table_flat, srcs, dsts, zfeat, ones):
    mesh = plsc.VectorSubcoreMesh(core_axis_name="c", subcore_axis_name="s")
    f = pl.kernel(
        _sc_body,
        out_type=[
            jax.ShapeDtypeStruct((6, NP, H), jnp.float32),
            jax.ShapeDtypeStruct((3, NP, H), jnp.float32),
        ],
        mesh=mesh,
        scratch_types=[
            pltpu.VMEM((GROUP, CHUNK), jnp.int32),
            pltpu.VMEM((CHUNK,), jnp.int32),
            pltpu.VMEM((CHUNK, H), jnp.float32),
            pltpu.VMEM_SHARED((NP, H), jnp.float32),
            pltpu.SemaphoreType.DMA,
            pltpu.SemaphoreType.DMA,
        ],
    )
    return f(table_flat, srcs, dsts, zfeat, ones)


def _dense_kernel(sums_ref, cnt_ref, cell_ref, Wl_ref, bl_ref, Wr_ref,
                  Wproj_ref, bproj_ref, Wout_ref, bout_ref, o_ref):
    cnt = jnp.max(cnt_ref[...], axis=-1)
    inv = 1.0 / jnp.maximum(cnt, 1.0)
    cell = cell_ref[...]
    Wl = Wl_ref[...]
    Wr = Wr_ref[...]
    bl = bl_ref[...]

    dot = functools.partial(jnp.dot, precision=lax.Precision.HIGHEST,
                            preferred_element_type=jnp.float32)

    def layer(i, cell, agg_base):
        u = dot(cell, jnp.sum(Wr[i], axis=0)) + jnp.sum(bl[i], axis=0)[None, :]
        for r in range(3):
            agg = sums_ref[agg_base + r] * inv[r][:, None]
            u = u + dot(agg, Wl[i, r])
        return jnp.maximum(u, 0.0)

    cell = layer(0, cell, 0)
    cell = layer(1, cell, 3)
    cell = layer(2, cell, 3)
    h = dot(cell, Wproj_ref[...]) + bproj_ref[...]
    sp = jnp.maximum(h, 0.0) + jnp.log(1.0 + jnp.exp(-jnp.abs(h)))
    o_ref[...] = dot(sp, Wout_ref[...]) + bout_ref[...]


def _dense_chain(sums, cnt, cell_pad, Wl, bl, Wr, Wproj, bproj, Wout, bout):
    blk = 632
    grid = (NP // blk,)
    return pl.pallas_call(
        _dense_kernel,
        grid=grid,
        in_specs=[
            pl.BlockSpec((6, blk, H), lambda i: (0, i, 0)),
            pl.BlockSpec((3, blk, H), lambda i: (0, i, 0)),
            pl.BlockSpec((blk, H), lambda i: (i, 0)),
            pl.BlockSpec((3, 3, H, H), lambda i: (0, 0, 0, 0)),
            pl.BlockSpec((3, 3, H), lambda i: (0, 0, 0)),
            pl.BlockSpec((3, 3, H, H), lambda i: (0, 0, 0, 0)),
            pl.BlockSpec((H, H), lambda i: (0, 0)),
            pl.BlockSpec((1, H), lambda i: (0, 0)),
            pl.BlockSpec((H, 1), lambda i: (0, 0)),
            pl.BlockSpec((1, 1), lambda i: (0, 0)),
        ],
        out_specs=pl.BlockSpec((blk, 1), lambda i: (i, 0)),
        out_shape=jax.ShapeDtypeStruct((NP, 1), jnp.float32),
    )(sums, cnt, cell_pad, Wl, bl, Wr, Wproj, bproj, Wout, bout)


def kernel(x_cell, x_atom, x_bond, x_motif, edge_index_atom, edge_index_bond,
           edge_index_motif, Wl, bl, Wr, Wproj, bproj, Wout, bout):
    eis = [edge_index_atom, edge_index_bond, edge_index_motif]
    pad = EPT - E
    ar = jnp.arange(pad, dtype=jnp.int32)
    pad_idx = N + (ar % 8)

    srcs, dsts = [], []
    for ei in eis:
        ei = ei.astype(jnp.int32)
        srcs.append(jnp.concatenate([ei[0], pad_idx]))
        dsts.append(jnp.concatenate([ei[1], pad_idx]))
    srcs3 = jnp.stack(srcs)
    dsts3 = jnp.stack(dsts).reshape(3, 16, CHUNKS_PER_TILE, CHUNK)
    offs = (jnp.arange(6, dtype=jnp.int32) * NP)[:, None]
    srcs6 = (jnp.concatenate([srcs3, srcs3]) + offs).reshape(
        6, 16, CHUNKS_PER_TILE, CHUNK)

    xpad = jnp.pad(jnp.stack([x_atom, x_bond, x_motif]),
                   ((0, 0), (0, NP - N), (0, 0)))
    table = _build_table(xpad).reshape(6 * NP, H)

    zfeat = jnp.zeros((CHUNK, H), jnp.float32)
    ones = jnp.ones((CHUNK, H), jnp.float32)

    sums, cnt = _sc_aggregate(table, srcs6, dsts3, zfeat, ones)

    cell_pad = jnp.pad(x_cell, ((0, NP - N), (0, 0)))
    out = _dense_chain(sums, cnt, cell_pad, Wl, bl, Wr,
                       Wproj, bproj.reshape(1, H), Wout, bout.reshape(1, 1))
    return out[:N]


# speedup vs baseline: 5.5792x; 1.3401x over previous
import functools

import jax
import jax.numpy as jnp
from jax import lax
from jax.experimental import pallas as pl
from jax.experimental.pallas import tpu as pltpu
from jax.experimental.pallas import tpu_sc as plsc

H = 128
N = 10000
E = 160000
NP = 10112
ROWS_PER_TILE = NP // 16
CHUNK = 128
GROUP = 16
GROUPS = 5
CHUNKS_PER_TILE = GROUP * GROUPS
EPT = 16 * CHUNKS_PER_TILE * CHUNK
N_PASS_PER_SC = 3


def _relu_table_kernel(x_ref, o_ref):
    p = pl.program_id(0)
    x = x_ref[...]
    o_ref[...] = jnp.where(p >= 3, jnp.maximum(x, 0.0), x)


def _build_table(xpad):
    blk = 632
    return pl.pallas_call(
        _relu_table_kernel,
        grid=(6, NP // blk),
        in_specs=[pl.BlockSpec((1, blk, H), lambda p, i: (lax.rem(p, 3), i, 0))],
        out_specs=pl.BlockSpec((1, blk, H), lambda p, i: (p, i, 0)),
        out_shape=jax.ShapeDtypeStruct((6, NP, H), jnp.float32),
    )(xpad)


def _row_chunks():
    out = []
    k = 0
    while k < ROWS_PER_TILE:
        out.append((k, min(CHUNK, ROWS_PER_TILE - k)))
        k += CHUNK
    return out


def _sc_body(table, srcs, dsts, zfeat, ones,
             out_sums, out_cnt,
             srcc0, srcc1, dstc0, dstc1, fbuf0, fbuf1, acc,
             isem0, isem1, gsem0, gsem1, ssem0, ssem1):
    cid = lax.axis_index("c")
    sid = lax.axis_index("s")
    row0 = sid * ROWS_PER_TILE
    srcc = [srcc0, srcc1]
    dstc = [dstc0, dstc1]
    fbuf = [fbuf0, fbuf1]
    isem = [isem0, isem1]
    gsem = [gsem0, gsem1]
    ssem = [ssem0, ssem1]

    def zero_acc():
        pltpu.sync_copy(zfeat, fbuf[0])
        for k, sz in _row_chunks():
            pltpu.sync_copy(fbuf[0].at[pl.ds(0, sz)],
                            acc.at[pl.ds(row0 + k, sz)])

    def copy_out(dst_hbm):
        for k, sz in _row_chunks():
            pltpu.sync_copy(acc.at[pl.ds(row0 + k, sz)],
                            fbuf[0].at[pl.ds(0, sz)])
            pltpu.sync_copy(fbuf[0].at[pl.ds(0, sz)],
                            dst_hbm.at[pl.ds(row0 + k, sz)])

    NB = 2
    HALF = CHUNKS_PER_TILE // NB

    for p_i in range(N_PASS_PER_SC):
        p = cid * N_PASS_PER_SC + p_i
        zero_acc()
        plsc.subcore_barrier()

        for b in range(NB):
            pltpu.async_copy(srcs.at[p, sid, b], srcc[b], isem[b])
            pltpu.async_copy(dsts.at[p_i, sid, b], dstc[b], isem[b])

        def _pair(t, carry):
            for b in range(NB):
                c = NB * t + b
                pltpu.make_async_copy(srcs.at[p, sid, c], srcc[b],
                                      isem[b]).wait()
                pltpu.make_async_copy(dsts.at[p_i, sid, c], dstc[b],
                                      isem[b]).wait()
                @pl.when(t > 0)
                def _():
                    pltpu.make_async_copy(fbuf[b], acc.at[dstc[b]],
                                          ssem[b]).wait()

                pltpu.async_copy(table.at[srcc[b]], fbuf[b], gsem[b]).wait()
                pltpu.async_copy(fbuf[b], acc.at[dstc[b]], ssem[b], add=True)

                @pl.when(t < HALF - 1)
                def _():
                    pltpu.async_copy(srcs.at[p, sid, c + NB], srcc[b],
                                     isem[b])
                    pltpu.async_copy(dsts.at[p_i, sid, c + NB], dstc[b],
                                     isem[b])

            return carry

        lax.fori_loop(0, HALF, _pair, 0)
        for b in range(NB):
            pltpu.make_async_copy(fbuf[b], acc.at[dstc[b]], ssem[b]).wait()
        plsc.subcore_barrier()
        copy_out(out_sums.at[p])
        plsc.subcore_barrier()

    def count_pass(r):
        zero_acc()
        plsc.subcore_barrier()
        pltpu.sync_copy(ones, fbuf[0])
        for b in range(NB):
            pltpu.async_copy(dsts.at[r, sid, b], dstc[b], isem[b])

        def _pair(t, carry):
            for b in range(NB):
                c = NB * t + b
                pltpu.make_async_copy(dsts.at[r, sid, c], dstc[b],
                                      isem[b]).wait()

                @pl.when(t > 0)
                def _():
                    pltpu.make_async_copy(fbuf[0], acc.at[dstc[b]],
                                          ssem[b]).wait()

                pltpu.async_copy(fbuf[0], acc.at[dstc[b]], ssem[b], add=True)

                @pl.when(t < HALF - 1)
                def _():
                    pltpu.async_copy(dsts.at[r, sid, c + NB], dstc[b],
                                     isem[b])

            return carry

        lax.fori_loop(0, HALF, _pair, 0)
        for b in range(NB):
            pltpu.make_async_copy(fbuf[0], acc.at[dstc[b]], ssem[b]).wait()
        plsc.subcore_barrier()
        copy_out(out_cnt.at[r])
        plsc.subcore_barrier()

    @pl.when(cid == 0)
    def _():
        count_pass(0)
        count_pass(1)

    @pl.when(cid == 1)
    def _():
        count_pass(2)


def _sc_aggregate(table_flat, srcs, dsts, zfeat, ones):
    mesh = plsc.VectorSubcoreMesh(core_axis_name="c", subcore_axis_name="s")
    f = pl.kernel(
        _sc_body,
        out_type=[
            jax.ShapeDtypeStruct((6, NP, H), jnp.float32),
            jax.ShapeDtypeStruct((3, NP, H), jnp.float32),
        ],
        mesh=mesh,
        scratch_types=[
            pltpu.VMEM((CHUNK,), jnp.int32),
            pltpu.VMEM((CHUNK,), jnp.int32),
            pltpu.VMEM((CHUNK,), jnp.int32),
            pltpu.VMEM((CHUNK,), jnp.int32),
            pltpu.VMEM((CHUNK, H), jnp.float32),
            pltpu.VMEM((CHUNK, H), jnp.float32),
            pltpu.VMEM_SHARED((NP, H), jnp.float32),
            pltpu.SemaphoreType.DMA,
            pltpu.SemaphoreType.DMA,
            pltpu.SemaphoreType.DMA,
            pltpu.SemaphoreType.DMA,
            pltpu.SemaphoreType.DMA,
            pltpu.SemaphoreType.DMA,
        ],
    )
    return f(table_flat, srcs, dsts, zfeat, ones)


def _dense_kernel(sums_ref, cnt_ref, cell_ref, Wl_ref, bl_ref, Wr_ref,
                  Wproj_ref, bproj_ref, Wout_ref, bout_ref, o_ref):
    cnt = jnp.max(cnt_ref[...], axis=-1)
    inv = 1.0 / jnp.maximum(cnt, 1.0)
    cell = cell_ref[...]
    Wl = Wl_ref[...]
    Wr = Wr_ref[...]
    bl = bl_ref[...]

    dot = functools.partial(jnp.dot, precision=lax.Precision.HIGHEST,
                            preferred_element_type=jnp.float32)

    def layer(i, cell, agg_base):
        u = dot(cell, jnp.sum(Wr[i], axis=0)) + jnp.sum(bl[i], axis=0)[None, :]
        for r in range(3):
            agg = sums_ref[agg_base + r] * inv[r][:, None]
            u = u + dot(agg, Wl[i, r])
        return jnp.maximum(u, 0.0)

    cell = layer(0, cell, 0)
    cell = layer(1, cell, 3)
    cell = layer(2, cell, 3)
    h = dot(cell, Wproj_ref[...]) + bproj_ref[...]
    sp = jnp.maximum(h, 0.0) + jnp.log(1.0 + jnp.exp(-jnp.abs(h)))
    o_ref[...] = dot(sp, Wout_ref[...]) + bout_ref[...]


def _dense_chain(sums, cnt, cell_pad, Wl, bl, Wr, Wproj, bproj, Wout, bout):
    blk = 632
    grid = (NP // blk,)
    return pl.pallas_call(
        _dense_kernel,
        grid=grid,
        in_specs=[
            pl.BlockSpec((6, blk, H), lambda i: (0, i, 0)),
            pl.BlockSpec((3, blk, H), lambda i: (0, i, 0)),
            pl.BlockSpec((blk, H), lambda i: (i, 0)),
            pl.BlockSpec((3, 3, H, H), lambda i: (0, 0, 0, 0)),
            pl.BlockSpec((3, 3, H), lambda i: (0, 0, 0)),
            pl.BlockSpec((3, 3, H, H), lambda i: (0, 0, 0, 0)),
            pl.BlockSpec((H, H), lambda i: (0, 0)),
            pl.BlockSpec((1, H), lambda i: (0, 0)),
            pl.BlockSpec((H, 1), lambda i: (0, 0)),
            pl.BlockSpec((1, 1), lambda i: (0, 0)),
        ],
        out_specs=pl.BlockSpec((blk, 1), lambda i: (i, 0)),
        out_shape=jax.ShapeDtypeStruct((NP, 1), jnp.float32),
    )(sums, cnt, cell_pad, Wl, bl, Wr, Wproj, bproj, Wout, bout)


def kernel(x_cell, x_atom, x_bond, x_motif, edge_index_atom, edge_index_bond,
           edge_index_motif, Wl, bl, Wr, Wproj, bproj, Wout, bout):
    eis = [edge_index_atom, edge_index_bond, edge_index_motif]
    pad = EPT - E
    ar = jnp.arange(pad, dtype=jnp.int32)
    pad_idx = N + (ar % 8)

    srcs, dsts = [], []
    for ei in eis:
        ei = ei.astype(jnp.int32)
        srcs.append(jnp.concatenate([ei[0], pad_idx]))
        dsts.append(jnp.concatenate([ei[1], pad_idx]))
    srcs3 = jnp.stack(srcs)
    dsts3 = jnp.stack(dsts).reshape(3, 16, CHUNKS_PER_TILE, CHUNK)
    offs = (jnp.arange(6, dtype=jnp.int32) * NP)[:, None]
    srcs6 = (jnp.concatenate([srcs3, srcs3]) + offs).reshape(
        6, 16, CHUNKS_PER_TILE, CHUNK)

    xpad = jnp.pad(jnp.stack([x_atom, x_bond, x_motif]),
                   ((0, 0), (0, NP - N), (0, 0)))
    table = _build_table(xpad).reshape(6 * NP, H)

    zfeat = jnp.zeros((CHUNK, H), jnp.float32)
    ones = jnp.ones((CHUNK, H), jnp.float32)

    sums, cnt = _sc_aggregate(table, srcs6, dsts3, zfeat, ones)

    cell_pad = jnp.pad(x_cell, ((0, NP - N), (0, 0)))
    out = _dense_chain(sums, cnt, cell_pad, Wl, bl, Wr,
                       Wproj, bproj.reshape(1, H), Wout, bout.reshape(1, 1))
    return out[:N]


# speedup vs baseline: 7.0643x; 1.2662x over previous
import functools

import jax
import jax.numpy as jnp
from jax import lax
from jax.experimental import pallas as pl
from jax.experimental.pallas import tpu as pltpu
from jax.experimental.pallas import tpu_sc as plsc

H = 128
N = 10000
E = 160000
NP = 10112
ROWS_PER_TILE = NP // 16
CHUNK = 128
CHUNKS_PER_TILE = 80
EPT = 16 * CHUNKS_PER_TILE * CHUNK
N_PASS_PER_SC = 3
NB = 2


def _relu_table_kernel(x_ref, o_ref):
    o_ref[...] = jnp.maximum(x_ref[...], 0.0)


def _build_relu_table(xpad):
    blk = 1264
    return pl.pallas_call(
        _relu_table_kernel,
        grid=(3, NP // blk),
        in_specs=[pl.BlockSpec((1, blk, H), lambda p, i: (p, i, 0))],
        out_specs=pl.BlockSpec((1, blk, H), lambda p, i: (p, i, 0)),
        out_shape=jax.ShapeDtypeStruct((3, NP, H), jnp.float32),
    )(xpad)


def _row_chunks():
    out = []
    k = 0
    while k < ROWS_PER_TILE:
        out.append((k, min(CHUNK, ROWS_PER_TILE - k)))
        k += CHUNK
    return out


def _sc_body(taba, tabb, srcs, dsts, zfeat, ones,
             out_sums, out_cnt,
             srcc0, srcc1, dstc0, dstc1, fbuf0, fbuf1, acc,
             isem0, isem1, gsem0, gsem1, ssem0, ssem1):
    cid = lax.axis_index("c")
    sid = lax.axis_index("s")
    row0 = sid * ROWS_PER_TILE
    srcc = [srcc0, srcc1]
    dstc = [dstc0, dstc1]
    fbuf = [fbuf0, fbuf1]
    isem = [isem0, isem1]
    gsem = [gsem0, gsem1]
    ssem = [ssem0, ssem1]

    def zero_acc():
        pltpu.sync_copy(zfeat, fbuf[0])
        for k, sz in _row_chunks():
            pltpu.async_copy(fbuf[0].at[pl.ds(0, sz)],
                             acc.at[pl.ds(row0 + k, sz)], gsem[0])
        for k, sz in _row_chunks():
            pltpu.make_async_copy(fbuf[0].at[pl.ds(0, sz)],
                                  acc.at[pl.ds(row0 + k, sz)],
                                  gsem[0]).wait()

    def copy_out(dst_hbm):
        chunks = _row_chunks()
        for i, (k, sz) in enumerate(chunks):
            b = i % NB
            if i >= NB:
                kp, szp = chunks[i - NB]
                pltpu.make_async_copy(fbuf[b].at[pl.ds(0, szp)],
                                      dst_hbm.at[pl.ds(row0 + kp, szp)],
                                      ssem[b]).wait()
            pltpu.async_copy(acc.at[pl.ds(row0 + k, sz)],
                             fbuf[b].at[pl.ds(0, sz)], gsem[b]).wait()
            pltpu.async_copy(fbuf[b].at[pl.ds(0, sz)],
                             dst_hbm.at[pl.ds(row0 + k, sz)], ssem[b])
        for i in range(len(chunks) - NB, len(chunks)):
            k, sz = chunks[i]
            pltpu.make_async_copy(fbuf[i % NB].at[pl.ds(0, sz)],
                                  dst_hbm.at[pl.ds(row0 + k, sz)],
                                  ssem[i % NB]).wait()

    HALF = CHUNKS_PER_TILE // NB

    for p_i in range(N_PASS_PER_SC):
        zero_acc()
        plsc.subcore_barrier()

        for b in range(NB):
            pltpu.async_copy(srcs.at[p_i, sid, b], srcc[b], isem[b])
            pltpu.async_copy(dsts.at[p_i, sid, b], dstc[b], isem[b])

        def _pair(t, carry):
            for b in range(NB):
                c = NB * t + b
                pltpu.make_async_copy(srcs.at[p_i, sid, c], srcc[b],
                                      isem[b]).wait()
                pltpu.make_async_copy(dsts.at[p_i, sid, c], dstc[b],
                                      isem[b]).wait()
                @pl.when(t > 0)
                def _():
                    pltpu.make_async_copy(fbuf[b], acc.at[dstc[b]],
                                          ssem[b]).wait()

                @pl.when(cid == 0)
                def _():
                    pltpu.async_copy(taba.at[srcc[b]], fbuf[b], gsem[b])

                @pl.when(cid == 1)
                def _():
                    pltpu.async_copy(tabb.at[srcc[b]], fbuf[b], gsem[b])

                pltpu.make_async_copy(taba.at[srcc[b]], fbuf[b],
                                      gsem[b]).wait()
                pltpu.async_copy(fbuf[b], acc.at[dstc[b]], ssem[b], add=True)

                @pl.when(t < HALF - 1)
                def _():
                    pltpu.async_copy(srcs.at[p_i, sid, c + NB], srcc[b],
                                     isem[b])
                    pltpu.async_copy(dsts.at[p_i, sid, c + NB], dstc[b],
                                     isem[b])

            return carry

        lax.fori_loop(0, HALF, _pair, 0)
        for b in range(NB):
            pltpu.make_async_copy(fbuf[b], acc.at[dstc[b]], ssem[b]).wait()
        plsc.subcore_barrier()
        p = cid * N_PASS_PER_SC + p_i
        copy_out(out_sums.at[p])
        plsc.subcore_barrier()

    def count_pass(r, slot, lo, hi):
        zero_acc()
        plsc.subcore_barrier()
        pltpu.sync_copy(ones, fbuf[0])
        for b in range(NB):
            pltpu.async_copy(dsts.at[r, sid, lo + b], dstc[b], isem[b])

        def _pair(t, carry):
            for b in range(NB):
                c = lo + NB * t + b
                pltpu.make_async_copy(dsts.at[r, sid, c], dstc[b],
                                      isem[b]).wait()

                @pl.when(t > 0)
                def _():
                    pltpu.make_async_copy(fbuf[0], acc.at[dstc[b]],
                                          ssem[b]).wait()

                pltpu.async_copy(fbuf[0], acc.at[dstc[b]], ssem[b], add=True)

                @pl.when(t < (hi - lo) // NB - 1)
                def _():
                    pltpu.async_copy(dsts.at[r, sid, c + NB], dstc[b],
                                     isem[b])

            return carry

        lax.fori_loop(0, (hi - lo) // NB, _pair, 0)
        for b in range(NB):
            pltpu.make_async_copy(fbuf[0], acc.at[dstc[b]], ssem[b]).wait()
        plsc.subcore_barrier()
        copy_out(out_cnt.at[slot])
        plsc.subcore_barrier()

    HC = CHUNKS_PER_TILE // 2

    @pl.when(cid == 0)
    def _():
        count_pass(0, 0, 0, CHUNKS_PER_TILE)
        count_pass(2, 2, 0, HC)

    @pl.when(cid == 1)
    def _():
        count_pass(1, 1, 0, CHUNKS_PER_TILE)
        count_pass(2, 3, HC, CHUNKS_PER_TILE)


def _sc_aggregate(taba, tabb, srcs, dsts, zfeat, ones):
    mesh = plsc.VectorSubcoreMesh(core_axis_name="c", subcore_axis_name="s")
    f = pl.kernel(
        _sc_body,
        out_type=[
            jax.ShapeDtypeStruct((6, NP, H), jnp.float32),
            jax.ShapeDtypeStruct((4, NP, H), jnp.float32),
        ],
        mesh=mesh,
        scratch_types=[
            pltpu.VMEM((CHUNK,), jnp.int32),
            pltpu.VMEM((CHUNK,), jnp.int32),
            pltpu.VMEM((CHUNK,), jnp.int32),
            pltpu.VMEM((CHUNK,), jnp.int32),
            pltpu.VMEM((CHUNK, H), jnp.float32),
            pltpu.VMEM((CHUNK, H), jnp.float32),
            pltpu.VMEM_SHARED((NP, H), jnp.float32),
            pltpu.SemaphoreType.DMA,
            pltpu.SemaphoreType.DMA,
            pltpu.SemaphoreType.DMA,
            pltpu.SemaphoreType.DMA,
            pltpu.SemaphoreType.DMA,
            pltpu.SemaphoreType.DMA,
        ],
    )
    return f(taba, tabb, srcs, dsts, zfeat, ones)


def _dense_kernel(sums_ref, cnt_ref, cell_ref, Wl_ref, bl_ref, Wr_ref,
                  Wproj_ref, bproj_ref, Wout_ref, bout_ref, o_ref):
    cmax = jnp.max(cnt_ref[...], axis=-1)
    cnt = jnp.stack([cmax[0], cmax[1], cmax[2] + cmax[3]])
    inv = 1.0 / jnp.maximum(cnt, 1.0)
    cell = cell_ref[...]
    Wl = Wl_ref[...]
    Wr = Wr_ref[...]
    bl = bl_ref[...]

    dot = functools.partial(jnp.dot, precision=lax.Precision.HIGHEST,
                            preferred_element_type=jnp.float32)

    def layer(i, cell, agg_base):
        a = jnp.concatenate(
            [sums_ref[agg_base + r] * inv[r][:, None] for r in range(3)]
            + [cell], axis=1)
        w = jnp.concatenate([Wl[i, 0], Wl[i, 1], Wl[i, 2],
                             jnp.sum(Wr[i], axis=0)], axis=0)
        u = dot(a, w) + jnp.sum(bl[i], axis=0)[None, :]
        return jnp.maximum(u, 0.0)

    cell = layer(0, cell, 0)
    cell = layer(1, cell, 3)
    cell = layer(2, cell, 3)
    h = dot(cell, Wproj_ref[...]) + bproj_ref[...]
    sp = jnp.maximum(h, 0.0) + jnp.log(1.0 + jnp.exp(-jnp.abs(h)))
    o_ref[...] = dot(sp, Wout_ref[...]) + bout_ref[...]


def _dense_chain(sums, cnt, cell_pad, Wl, bl, Wr, Wproj, bproj, Wout, bout):
    blk = 2528
    grid = (NP // blk,)
    return pl.pallas_call(
        _dense_kernel,
        grid=grid,
        in_specs=[
            pl.BlockSpec((6, blk, H), lambda i: (0, i, 0)),
            pl.BlockSpec((4, blk, H), lambda i: (0, i, 0)),
            pl.BlockSpec((blk, H), lambda i: (i, 0)),
            pl.BlockSpec((3, 3, H, H), lambda i: (0, 0, 0, 0)),
            pl.BlockSpec((3, 3, H), lambda i: (0, 0, 0)),
            pl.BlockSpec((3, 3, H, H), lambda i: (0, 0, 0, 0)),
            pl.BlockSpec((H, H), lambda i: (0, 0)),
            pl.BlockSpec((1, H), lambda i: (0, 0)),
            pl.BlockSpec((H, 1), lambda i: (0, 0)),
            pl.BlockSpec((1, 1), lambda i: (0, 0)),
        ],
        out_specs=pl.BlockSpec((blk, 1), lambda i: (i, 0)),
        out_shape=jax.ShapeDtypeStruct((NP, 1), jnp.float32),
    )(sums, cnt, cell_pad, Wl, bl, Wr, Wproj, bproj, Wout, bout)


def kernel(x_cell, x_atom, x_bond, x_motif, edge_index_atom, edge_index_bond,
           edge_index_motif, Wl, bl, Wr, Wproj, bproj, Wout, bout):
    eis = [edge_index_atom, edge_index_bond, edge_index_motif]
    pad = EPT - E
    ar = jnp.arange(pad, dtype=jnp.int32)
    pad_idx = N + (ar % 8)

    srcs, dsts = [], []
    for ei in eis:
        ei = ei.astype(jnp.int32)
        srcs.append(jnp.concatenate([ei[0], pad_idx]))
        dsts.append(jnp.concatenate([ei[1], pad_idx]))
    offs = (jnp.arange(3, dtype=jnp.int32) * NP)[:, None]
    srcs3 = (jnp.stack(srcs) + offs).reshape(3, 16, CHUNKS_PER_TILE, CHUNK)
    dsts3 = jnp.stack(dsts).reshape(3, 16, CHUNKS_PER_TILE, CHUNK)

    xpad = jnp.pad(jnp.stack([x_atom, x_bond, x_motif]),
                   ((0, 0), (0, NP - N), (0, 0)))
    taba = xpad.reshape(3 * NP, H)
    tabb = _build_relu_table(xpad).reshape(3 * NP, H)

    zfeat = jnp.zeros((CHUNK, H), jnp.float32)
    ones = jnp.ones((CHUNK, H), jnp.float32)

    sums, cnt = _sc_aggregate(taba, tabb, srcs3, dsts3, zfeat, ones)

    cell_pad = jnp.pad(x_cell, ((0, NP - N), (0, 0)))
    out = _dense_chain(sums, cnt, cell_pad, Wl, bl, Wr,
                       Wproj, bproj.reshape(1, H), Wout, bout.reshape(1, 1))
    return out[:N]


# speedup vs baseline: 8.0113x; 1.1341x over previous
import functools

import jax
import jax.numpy as jnp
from jax import lax
from jax.experimental import pallas as pl
from jax.experimental.pallas import tpu as pltpu
from jax.experimental.pallas import tpu_sc as plsc

H = 128
N = 10000
E = 160000
NP = 10112
ROWS_PER_TILE = NP // 16
EDGES_PER_TILE = E // 16
CHUNK = 128
FULL_CHUNKS = EDGES_PER_TILE // CHUNK
TAIL = EDGES_PER_TILE - FULL_CHUNKS * CHUNK
N_PASS_PER_SC = 3
NB = 2


def _relu_table_kernel(x_ref, o_ref):
    o_ref[...] = jnp.maximum(x_ref[...], 0.0)


def _build_relu_table(xs):
    blk = 2000
    return pl.pallas_call(
        _relu_table_kernel,
        grid=(3, N // blk),
        in_specs=[pl.BlockSpec((1, blk, H), lambda p, i: (p, i, 0))],
        out_specs=pl.BlockSpec((1, blk, H), lambda p, i: (p, i, 0)),
        out_shape=jax.ShapeDtypeStruct((3, N, H), jnp.float32),
    )(xs)


def _row_chunks():
    out = []
    k = 0
    while k < ROWS_PER_TILE:
        out.append((k, min(CHUNK, ROWS_PER_TILE - k)))
        k += CHUNK
    return out


def _sc_body(ta0, ta1, ta2, tabb, s0, s1, s2, d0, d1, d2, zfeat, ones,
             out_sums, out_cnt,
             srcc0, srcc1, srcc2, srcc3, srcc4, srcc5,
             dstc0, dstc1, dstc2, dstc3, dstc4, dstc5,
             srcct, dstct, ftail,
             fbuf0, fbuf1, acc,
             isem0, isem1, isem2, isem3, isem4, isem5,
             gsem0, gsem1, ssem0, ssem1):
    cid = lax.axis_index("c")
    sid = lax.axis_index("s")
    row0 = sid * ROWS_PER_TILE
    ebase = sid * EDGES_PER_TILE
    ta = [ta0, ta1, ta2]
    sall = [s0, s1, s2]
    dall = [d0, d1, d2]
    srcc = [srcc0, srcc1, srcc2, srcc3, srcc4, srcc5]
    dstc = [dstc0, dstc1, dstc2, dstc3, dstc4, dstc5]
    fbuf = [fbuf0, fbuf1]
    isem = [isem0, isem1, isem2, isem3, isem4, isem5]
    gsem = [gsem0, gsem1]
    ssem = [ssem0, ssem1]

    def zero_acc():
        pltpu.sync_copy(zfeat, fbuf[0])
        for k, sz in _row_chunks():
            pltpu.async_copy(fbuf[0].at[pl.ds(0, sz)],
                             acc.at[pl.ds(row0 + k, sz)], gsem[0])
        for k, sz in _row_chunks():
            pltpu.make_async_copy(fbuf[0].at[pl.ds(0, sz)],
                                  acc.at[pl.ds(row0 + k, sz)],
                                  gsem[0]).wait()

    def copy_out(dst_hbm):
        chunks = _row_chunks()
        for i, (k, sz) in enumerate(chunks):
            b = i % NB
            if i >= NB:
                kp, szp = chunks[i - NB]
                pltpu.make_async_copy(fbuf[b].at[pl.ds(0, szp)],
                                      dst_hbm.at[pl.ds(row0 + kp, szp)],
                                      ssem[b]).wait()
            pltpu.async_copy(acc.at[pl.ds(row0 + k, sz)],
                             fbuf[b].at[pl.ds(0, sz)], gsem[b]).wait()
            pltpu.async_copy(fbuf[b].at[pl.ds(0, sz)],
                             dst_hbm.at[pl.ds(row0 + k, sz)], ssem[b])
        for i in range(len(chunks) - NB, len(chunks)):
            k, sz = chunks[i]
            pltpu.make_async_copy(fbuf[i % NB].at[pl.ds(0, sz)],
                                  dst_hbm.at[pl.ds(row0 + k, sz)],
                                  ssem[i % NB]).wait()

    NI = 6
    TRIPS = FULL_CHUNKS // NI

    for p_i in range(N_PASS_PER_SC):
        src1d = sall[p_i]
        dst1d = dall[p_i]
        zero_acc()
        plsc.subcore_barrier()

        for u in range(4):
            pltpu.async_copy(src1d.at[pl.ds(ebase + u * CHUNK, CHUNK)],
                             srcc[u], isem[u])
            pltpu.async_copy(dst1d.at[pl.ds(ebase + u * CHUNK, CHUNK)],
                             dstc[u], isem[u])

        def _six(t, carry):
            for u in range(NI):
                b = u % NB
                c = NI * t + u
                off = ebase + c * CHUNK
                pltpu.make_async_copy(src1d.at[pl.ds(off, CHUNK)], srcc[u],
                                      isem[u]).wait()
                pltpu.make_async_copy(dst1d.at[pl.ds(off, CHUNK)], dstc[u],
                                      isem[u]).wait()

                def _drain():
                    pltpu.make_async_copy(fbuf[b],
                                          acc.at[dstc[(u + 4) % NI]],
                                          ssem[b]).wait()
                if u < NB:
                    @pl.when(t > 0)
                    def _():
                        _drain()
                else:
                    _drain()

                @pl.when(cid == 0)
                def _():
                    pltpu.async_copy(ta[p_i].at[srcc[u]], fbuf[b], gsem[b])

                @pl.when(cid == 1)
                def _():
                    pltpu.async_copy(tabb.at[p_i].at[srcc[u]], fbuf[b],
                                     gsem[b])

                pltpu.make_async_copy(ta[p_i].at[srcc[u]], fbuf[b],
                                      gsem[b]).wait()
                pltpu.async_copy(fbuf[b], acc.at[dstc[u]], ssem[b], add=True)

                def _pref():
                    off2 = ebase + (c + 4) * CHUNK
                    pltpu.async_copy(src1d.at[pl.ds(off2, CHUNK)],
                                     srcc[(u + 4) % NI], isem[(u + 4) % NI])
                    pltpu.async_copy(dst1d.at[pl.ds(off2, CHUNK)],
                                     dstc[(u + 4) % NI], isem[(u + 4) % NI])
                if u < NB:
                    _pref()
                else:
                    @pl.when(t < TRIPS - 1)
                    def _():
                        _pref()

            return carry

        lax.fori_loop(0, TRIPS, _six, 0)
        for b in range(NB):
            pltpu.make_async_copy(fbuf[b], acc.at[dstc[b]], ssem[b]).wait()

        toff = ebase + FULL_CHUNKS * CHUNK
        pltpu.sync_copy(src1d.at[pl.ds(toff, TAIL)], srcct)
        pltpu.sync_copy(dst1d.at[pl.ds(toff, TAIL)], dstct)

        @pl.when(cid == 0)
        def _():
            pltpu.async_copy(ta[p_i].at[srcct], ftail, gsem[0])

        @pl.when(cid == 1)
        def _():
            pltpu.async_copy(tabb.at[p_i].at[srcct], ftail, gsem[0])

        pltpu.make_async_copy(ta[p_i].at[srcct], ftail, gsem[0]).wait()
        pltpu.sync_copy(ftail, acc.at[dstct], add=True)

        plsc.subcore_barrier()
        p = cid * N_PASS_PER_SC + p_i
        copy_out(out_sums.at[p])
        plsc.subcore_barrier()

    def count_pass(r, slot, lo, hi, with_tail):
        NI = 6
        trips = (hi - lo) // NI
        dst1d = dall[r]
        zero_acc()
        plsc.subcore_barrier()
        pltpu.sync_copy(ones, fbuf[0])
        for u in range(4):
            pltpu.async_copy(dst1d.at[pl.ds(ebase + (lo + u) * CHUNK, CHUNK)],
                             dstc[u], isem[u])

        def _six(t, carry):
            for u in range(NI):
                b = u % NB
                c = lo + NI * t + u
                off = ebase + c * CHUNK
                pltpu.make_async_copy(dst1d.at[pl.ds(off, CHUNK)], dstc[u],
                                      isem[u]).wait()

                def _drain():
                    pltpu.make_async_copy(fbuf[0], acc.at[dstc[u]],
                                          ssem[b]).wait()
                if u < NB:
                    @pl.when(t > 0)
                    def _():
                        _drain()
                else:
                    _drain()

                pltpu.async_copy(fbuf[0], acc.at[dstc[u]], ssem[b], add=True)

                def _pref():
                    off2 = ebase + (c + 4) * CHUNK
                    pltpu.async_copy(dst1d.at[pl.ds(off2, CHUNK)],
                                     dstc[(u + 4) % NI], isem[(u + 4) % NI])
                if u < NB:
                    _pref()
                else:
                    @pl.when(t < trips - 1)
                    def _():
                        _pref()

            return carry

        lax.fori_loop(0, trips, _six, 0)
        for b in range(NB):
            pltpu.make_async_copy(fbuf[0], acc.at[dstc[b]], ssem[b]).wait()

        if with_tail:
            toff = ebase + FULL_CHUNKS * CHUNK
            pltpu.sync_copy(dst1d.at[pl.ds(toff, TAIL)], dstct)
            pltpu.sync_copy(ones.at[pl.ds(0, TAIL)], ftail)
            pltpu.sync_copy(ftail, acc.at[dstct], add=True)

        plsc.subcore_barrier()
        copy_out(out_cnt.at[slot])
        plsc.subcore_barrier()

    HC = 36

    @pl.when(cid == 0)
    def _():
        count_pass(0, 0, 0, FULL_CHUNKS, True)
        count_pass(2, 2, 0, HC, False)

    @pl.when(cid == 1)
    def _():
        count_pass(1, 1, 0, FULL_CHUNKS, True)
        count_pass(2, 3, HC, FULL_CHUNKS, True)


def _sc_aggregate(tas, tabb, ss, ds, zfeat, ones):
    mesh = plsc.VectorSubcoreMesh(core_axis_name="c", subcore_axis_name="s")
    f = pl.kernel(
        _sc_body,
        out_type=[
            jax.ShapeDtypeStruct((6, NP, H), jnp.float32),
            jax.ShapeDtypeStruct((4, NP, H), jnp.float32),
        ],
        mesh=mesh,
        scratch_types=(
            [pltpu.VMEM((CHUNK,), jnp.int32)] * 12 +
            [
                pltpu.VMEM((TAIL,), jnp.int32),
                pltpu.VMEM((TAIL,), jnp.int32),
                pltpu.VMEM((TAIL, H), jnp.float32),
                pltpu.VMEM((CHUNK, H), jnp.float32),
                pltpu.VMEM((CHUNK, H), jnp.float32),
                pltpu.VMEM_SHARED((NP, H), jnp.float32),
            ] +
            [pltpu.SemaphoreType.DMA] * 10
        ),
    )
    return f(tas[0], tas[1], tas[2], tabb, ss[0], ss[1], ss[2],
             ds[0], ds[1], ds[2], zfeat, ones)


def _dense_kernel(sums_ref, cnt_ref, cell_ref, Wl_ref, bl_ref, Wr_ref,
                  Wproj_ref, bproj_ref, Wout_ref, bout_ref, o_ref):
    cmax = jnp.max(cnt_ref[...], axis=-1)
    cnt = jnp.stack([cmax[0], cmax[1], cmax[2] + cmax[3]])
    inv = 1.0 / jnp.maximum(cnt, 1.0)
    cell = cell_ref[...]
    Wl = Wl_ref[...]
    Wr = Wr_ref[...]
    bl = bl_ref[...]

    bdot = functools.partial(jnp.dot, preferred_element_type=jnp.float32)

    def dot3(a, w):
        ah = a.astype(jnp.bfloat16)
        al = (a - ah.astype(jnp.float32)).astype(jnp.bfloat16)
        wh = w.astype(jnp.bfloat16)
        wl = (w - wh.astype(jnp.float32)).astype(jnp.bfloat16)
        return bdot(ah, wh) + (bdot(ah, wl) + bdot(al, wh))

    def layer(i, cell, agg_base):
        a = jnp.concatenate(
            [sums_ref[agg_base + r] * inv[r][:, None] for r in range(3)]
            + [cell], axis=1)
        w = jnp.concatenate([Wl[i, 0], Wl[i, 1], Wl[i, 2],
                             jnp.sum(Wr[i], axis=0)], axis=0)
        u = dot3(a, w) + jnp.sum(bl[i], axis=0)[None, :]
        return jnp.maximum(u, 0.0)

    cell = layer(0, cell, 0)
    cell = layer(1, cell, 3)
    cell = layer(2, cell, 3)
    h = dot3(cell, Wproj_ref[...]) + bproj_ref[...]
    sp = jnp.maximum(h, 0.0) + jnp.log(1.0 + jnp.exp(-jnp.abs(h)))
    o_ref[...] = jnp.dot(sp, Wout_ref[...],
                         precision=lax.Precision.HIGHEST,
                         preferred_element_type=jnp.float32) + bout_ref[...]


def _dense_chain(sums, cnt, cell_pad, Wl, bl, Wr, Wproj, bproj, Wout, bout):
    blk = 2528
    grid = (NP // blk,)
    return pl.pallas_call(
        _dense_kernel,
        grid=grid,
        in_specs=[
            pl.BlockSpec((6, blk, H), lambda i: (0, i, 0)),
            pl.BlockSpec((4, blk, H), lambda i: (0, i, 0)),
            pl.BlockSpec((blk, H), lambda i: (i, 0)),
            pl.BlockSpec((3, 3, H, H), lambda i: (0, 0, 0, 0)),
            pl.BlockSpec((3, 3, H), lambda i: (0, 0, 0)),
            pl.BlockSpec((3, 3, H, H), lambda i: (0, 0, 0, 0)),
            pl.BlockSpec((H, H), lambda i: (0, 0)),
            pl.BlockSpec((1, H), lambda i: (0, 0)),
            pl.BlockSpec((H, 1), lambda i: (0, 0)),
            pl.BlockSpec((1, 1), lambda i: (0, 0)),
        ],
        out_specs=pl.BlockSpec((blk, 1), lambda i: (i, 0)),
        out_shape=jax.ShapeDtypeStruct((NP, 1), jnp.float32),
    )(sums, cnt, cell_pad, Wl, bl, Wr, Wproj, bproj, Wout, bout)


def kernel(x_cell, x_atom, x_bond, x_motif, edge_index_atom, edge_index_bond,
           edge_index_motif, Wl, bl, Wr, Wproj, bproj, Wout, bout):
    eis = [edge_index_atom, edge_index_bond, edge_index_motif]
    ss = [ei[0].astype(jnp.int32) for ei in eis]
    ds = [ei[1].astype(jnp.int32) for ei in eis]
    tas = [x_atom, x_bond, x_motif]
    tabb = _build_relu_table(jnp.stack(tas))

    zfeat = jnp.zeros((CHUNK, H), jnp.float32)
    ones = jnp.ones((CHUNK, H), jnp.float32)

    sums, cnt = _sc_aggregate(tas, tabb, ss, ds, zfeat, ones)

    cell_pad = jnp.pad(x_cell, ((0, NP - N), (0, 0)))
    out = _dense_chain(sums, cnt, cell_pad, Wl, bl, Wr,
                       Wproj, bproj.reshape(1, H), Wout, bout.reshape(1, 1))
    return out[:N]


# speedup vs baseline: 8.1832x; 1.0215x over previous
import functools

import jax
import jax.numpy as jnp
from jax import lax
from jax.experimental import pallas as pl
from jax.experimental.pallas import tpu as pltpu
from jax.experimental.pallas import tpu_sc as plsc

H = 128
N = 10000
E = 160000
NP = 10112
ROWS_PER_TILE = NP // 16
EDGES_PER_TILE = E // 16
CHUNK = 128
FULL_CHUNKS = EDGES_PER_TILE // CHUNK
TAIL = EDGES_PER_TILE - FULL_CHUNKS * CHUNK
N_PASS_PER_SC = 3
NB = 2


def _relu_table_kernel(x0, x1, x2, o0, o1, o2):
    o0[...] = jnp.maximum(x0[...], 0.0)
    o1[...] = jnp.maximum(x1[...], 0.0)
    o2[...] = jnp.maximum(x2[...], 0.0)


def _build_relu_table(xs):
    blk = 2000
    spec = pl.BlockSpec((blk, H), lambda i: (i, 0))
    return pl.pallas_call(
        _relu_table_kernel,
        grid=(N // blk,),
        in_specs=[spec] * 3,
        out_specs=[spec] * 3,
        out_shape=[jax.ShapeDtypeStruct((N, H), jnp.float32)] * 3,
    )(*xs)


def _row_chunks():
    out = []
    k = 0
    while k < ROWS_PER_TILE:
        out.append((k, min(CHUNK, ROWS_PER_TILE - k)))
        k += CHUNK
    return out


def _sc_body(ta0, ta1, ta2, tb0, tb1, tb2, s0, s1, s2, d0, d1, d2,
             zfeat, ones,
             out_sums, out_cnt,
             srcc0, srcc1, srcc2, srcc3, srcc4, srcc5,
             dstc0, dstc1, dstc2, dstc3, dstc4, dstc5,
             srcct, dstct, ftail,
             fbuf0, fbuf1, acc,
             isem0, isem1, isem2, isem3, isem4, isem5,
             gsem0, gsem1, ssem0, ssem1):
    cid = lax.axis_index("c")
    sid = lax.axis_index("s")
    row0 = sid * ROWS_PER_TILE
    ebase = sid * EDGES_PER_TILE
    ta = [ta0, ta1, ta2]
    tb = [tb0, tb1, tb2]
    sall = [s0, s1, s2]
    dall = [d0, d1, d2]
    srcc = [srcc0, srcc1, srcc2, srcc3, srcc4, srcc5]
    dstc = [dstc0, dstc1, dstc2, dstc3, dstc4, dstc5]
    fbuf = [fbuf0, fbuf1]
    isem = [isem0, isem1, isem2, isem3, isem4, isem5]
    gsem = [gsem0, gsem1]
    ssem = [ssem0, ssem1]

    def zero_acc():
        pltpu.sync_copy(zfeat, fbuf[0])
        for k, sz in _row_chunks():
            pltpu.async_copy(fbuf[0].at[pl.ds(0, sz)],
                             acc.at[pl.ds(row0 + k, sz)], gsem[0])
        for k, sz in _row_chunks():
            pltpu.make_async_copy(fbuf[0].at[pl.ds(0, sz)],
                                  acc.at[pl.ds(row0 + k, sz)],
                                  gsem[0]).wait()

    def copy_out(dst_hbm):
        chunks = _row_chunks()
        for i, (k, sz) in enumerate(chunks):
            b = i % NB
            if i >= NB:
                kp, szp = chunks[i - NB]
                pltpu.make_async_copy(fbuf[b].at[pl.ds(0, szp)],
                                      dst_hbm.at[pl.ds(row0 + kp, szp)],
                                      ssem[b]).wait()
            pltpu.async_copy(acc.at[pl.ds(row0 + k, sz)],
                             fbuf[b].at[pl.ds(0, sz)], gsem[b]).wait()
            pltpu.async_copy(fbuf[b].at[pl.ds(0, sz)],
                             dst_hbm.at[pl.ds(row0 + k, sz)], ssem[b])
        for i in range(len(chunks) - NB, len(chunks)):
            k, sz = chunks[i]
            pltpu.make_async_copy(fbuf[i % NB].at[pl.ds(0, sz)],
                                  dst_hbm.at[pl.ds(row0 + k, sz)],
                                  ssem[i % NB]).wait()

    NI = 6
    TRIPS = FULL_CHUNKS // NI

    for p_i in range(N_PASS_PER_SC):
        src1d = sall[p_i]
        dst1d = dall[p_i]
        zero_acc()
        plsc.subcore_barrier()

        for u in range(4):
            pltpu.async_copy(src1d.at[pl.ds(ebase + u * CHUNK, CHUNK)],
                             srcc[u], isem[u])
            pltpu.async_copy(dst1d.at[pl.ds(ebase + u * CHUNK, CHUNK)],
                             dstc[u], isem[u])

        def _six(t, carry):
            for u in range(NI):
                b = u % NB
                c = NI * t + u
                off = ebase + c * CHUNK
                pltpu.make_async_copy(src1d.at[pl.ds(off, CHUNK)], srcc[u],
                                      isem[u]).wait()
                pltpu.make_async_copy(dst1d.at[pl.ds(off, CHUNK)], dstc[u],
                                      isem[u]).wait()

                def _drain():
                    pltpu.make_async_copy(fbuf[b],
                                          acc.at[dstc[(u + 4) % NI]],
                                          ssem[b]).wait()
                if u < NB:
                    @pl.when(t > 0)
                    def _():
                        _drain()
                else:
                    _drain()

                @pl.when(cid == 0)
                def _():
                    pltpu.async_copy(ta[p_i].at[srcc[u]], fbuf[b], gsem[b])

                @pl.when(cid == 1)
                def _():
                    pltpu.async_copy(tb[p_i].at[srcc[u]], fbuf[b],
                                     gsem[b])

                pltpu.make_async_copy(ta[p_i].at[srcc[u]], fbuf[b],
                                      gsem[b]).wait()
                pltpu.async_copy(fbuf[b], acc.at[dstc[u]], ssem[b], add=True)

                def _pref():
                    off2 = ebase + (c + 4) * CHUNK
                    pltpu.async_copy(src1d.at[pl.ds(off2, CHUNK)],
                                     srcc[(u + 4) % NI], isem[(u + 4) % NI])
                    pltpu.async_copy(dst1d.at[pl.ds(off2, CHUNK)],
                                     dstc[(u + 4) % NI], isem[(u + 4) % NI])
                if u < NB:
                    _pref()
                else:
                    @pl.when(t < TRIPS - 1)
                    def _():
                        _pref()

            return carry

        lax.fori_loop(0, TRIPS, _six, 0)
        for b in range(NB):
            pltpu.make_async_copy(fbuf[b], acc.at[dstc[b]], ssem[b]).wait()

        toff = ebase + FULL_CHUNKS * CHUNK
        pltpu.sync_copy(src1d.at[pl.ds(toff, TAIL)], srcct)
        pltpu.sync_copy(dst1d.at[pl.ds(toff, TAIL)], dstct)

        @pl.when(cid == 0)
        def _():
            pltpu.async_copy(ta[p_i].at[srcct], ftail, gsem[0])

        @pl.when(cid == 1)
        def _():
            pltpu.async_copy(tb[p_i].at[srcct], ftail, gsem[0])

        pltpu.make_async_copy(ta[p_i].at[srcct], ftail, gsem[0]).wait()
        pltpu.sync_copy(ftail, acc.at[dstct], add=True)

        plsc.subcore_barrier()
        p = cid * N_PASS_PER_SC + p_i
        copy_out(out_sums.at[p])
        plsc.subcore_barrier()

    def count_pass(r, slot, lo, hi, with_tail):
        NI = 6
        trips = (hi - lo) // NI
        dst1d = dall[r]
        zero_acc()
        plsc.subcore_barrier()
        pltpu.sync_copy(ones, fbuf[0])
        for u in range(4):
            pltpu.async_copy(dst1d.at[pl.ds(ebase + (lo + u) * CHUNK, CHUNK)],
                             dstc[u], isem[u])

        def _six(t, carry):
            for u in range(NI):
                b = u % NB
                c = lo + NI * t + u
                off = ebase + c * CHUNK
                pltpu.make_async_copy(dst1d.at[pl.ds(off, CHUNK)], dstc[u],
                                      isem[u]).wait()

                def _drain():
                    pltpu.make_async_copy(fbuf[0], acc.at[dstc[u]],
                                          ssem[b]).wait()
                if u < NB:
                    @pl.when(t > 0)
                    def _():
                        _drain()
                else:
                    _drain()

                pltpu.async_copy(fbuf[0], acc.at[dstc[u]], ssem[b], add=True)

                def _pref():
                    off2 = ebase + (c + 4) * CHUNK
                    pltpu.async_copy(dst1d.at[pl.ds(off2, CHUNK)],
                                     dstc[(u + 4) % NI], isem[(u + 4) % NI])
                if u < NB:
                    _pref()
                else:
                    @pl.when(t < trips - 1)
                    def _():
                        _pref()

            return carry

        lax.fori_loop(0, trips, _six, 0)
        for b in range(NB):
            pltpu.make_async_copy(fbuf[0], acc.at[dstc[b]], ssem[b]).wait()

        if with_tail:
            toff = ebase + FULL_CHUNKS * CHUNK
            pltpu.sync_copy(dst1d.at[pl.ds(toff, TAIL)], dstct)
            pltpu.sync_copy(ones.at[pl.ds(0, TAIL)], ftail)
            pltpu.sync_copy(ftail, acc.at[dstct], add=True)

        plsc.subcore_barrier()
        copy_out(out_cnt.at[slot])
        plsc.subcore_barrier()

    HC = 36

    @pl.when(cid == 0)
    def _():
        count_pass(0, 0, 0, FULL_CHUNKS, True)
        count_pass(2, 2, 0, HC, False)

    @pl.when(cid == 1)
    def _():
        count_pass(1, 1, 0, FULL_CHUNKS, True)
        count_pass(2, 3, HC, FULL_CHUNKS, True)


def _sc_aggregate(tas, tbs, ss, ds, zfeat, ones):
    mesh = plsc.VectorSubcoreMesh(core_axis_name="c", subcore_axis_name="s")
    f = pl.kernel(
        _sc_body,
        out_type=[
            jax.ShapeDtypeStruct((6, NP, H), jnp.float32),
            jax.ShapeDtypeStruct((4, NP, H), jnp.float32),
        ],
        mesh=mesh,
        scratch_types=(
            [pltpu.VMEM((CHUNK,), jnp.int32)] * 12 +
            [
                pltpu.VMEM((TAIL,), jnp.int32),
                pltpu.VMEM((TAIL,), jnp.int32),
                pltpu.VMEM((TAIL, H), jnp.float32),
                pltpu.VMEM((CHUNK, H), jnp.float32),
                pltpu.VMEM((CHUNK, H), jnp.float32),
                pltpu.VMEM_SHARED((NP, H), jnp.float32),
            ] +
            [pltpu.SemaphoreType.DMA] * 10
        ),
    )
    return f(tas[0], tas[1], tas[2], tbs[0], tbs[1], tbs[2],
             ss[0], ss[1], ss[2], ds[0], ds[1], ds[2], zfeat, ones)


def _dense_kernel(sums_ref, cnt_ref, cell_ref, Wl_ref, bl_ref, Wr_ref,
                  Wproj_ref, bproj_ref, Wout_ref, bout_ref, o_ref):
    cmax = jnp.max(cnt_ref[...], axis=-1)
    cnt = jnp.stack([cmax[0], cmax[1], cmax[2] + cmax[3]])
    inv = 1.0 / jnp.maximum(cnt, 1.0)
    cell = cell_ref[...]
    Wl = Wl_ref[...]
    Wr = Wr_ref[...]
    bl = bl_ref[...]

    bdot = functools.partial(jnp.dot, preferred_element_type=jnp.float32)

    def dot3(a, w):
        ah = a.astype(jnp.bfloat16)
        al = (a - ah.astype(jnp.float32)).astype(jnp.bfloat16)
        wh = w.astype(jnp.bfloat16)
        wl = (w - wh.astype(jnp.float32)).astype(jnp.bfloat16)
        return bdot(ah, wh) + (bdot(ah, wl) + bdot(al, wh))

    def layer(i, cell, agg_base):
        a = jnp.concatenate(
            [sums_ref[agg_base + r] * inv[r][:, None] for r in range(3)]
            + [cell], axis=1)
        w = jnp.concatenate([Wl[i, 0], Wl[i, 1], Wl[i, 2],
                             jnp.sum(Wr[i], axis=0)], axis=0)
        u = dot3(a, w) + jnp.sum(bl[i], axis=0)[None, :]
        return jnp.maximum(u, 0.0)

    cell = layer(0, cell, 0)
    cell = layer(1, cell, 3)
    cell = layer(2, cell, 3)
    h = dot3(cell, Wproj_ref[...]) + bproj_ref[...]
    sp = jnp.maximum(h, 0.0) + jnp.log(1.0 + jnp.exp(-jnp.abs(h)))
    o_ref[...] = jnp.dot(sp, Wout_ref[...],
                         precision=lax.Precision.HIGHEST,
                         preferred_element_type=jnp.float32) + bout_ref[...]


def _dense_chain(sums, cnt, cell_pad, Wl, bl, Wr, Wproj, bproj, Wout, bout):
    blk = 1264
    grid = (NP // blk,)
    return pl.pallas_call(
        _dense_kernel,
        grid=grid,
        in_specs=[
            pl.BlockSpec((6, blk, H), lambda i: (0, i, 0)),
            pl.BlockSpec((4, blk, H), lambda i: (0, i, 0)),
            pl.BlockSpec((blk, H), lambda i: (i, 0)),
            pl.BlockSpec((3, 3, H, H), lambda i: (0, 0, 0, 0)),
            pl.BlockSpec((3, 3, H), lambda i: (0, 0, 0)),
            pl.BlockSpec((3, 3, H, H), lambda i: (0, 0, 0, 0)),
            pl.BlockSpec((H, H), lambda i: (0, 0)),
            pl.BlockSpec((1, H), lambda i: (0, 0)),
            pl.BlockSpec((H, 1), lambda i: (0, 0)),
            pl.BlockSpec((1, 1), lambda i: (0, 0)),
        ],
        out_specs=pl.BlockSpec((blk, 1), lambda i: (i, 0)),
        out_shape=jax.ShapeDtypeStruct((NP, 1), jnp.float32),
    )(sums, cnt, cell_pad, Wl, bl, Wr, Wproj, bproj, Wout, bout)


def kernel(x_cell, x_atom, x_bond, x_motif, edge_index_atom, edge_index_bond,
           edge_index_motif, Wl, bl, Wr, Wproj, bproj, Wout, bout):
    eis = [edge_index_atom, edge_index_bond, edge_index_motif]
    ss = [ei[0].astype(jnp.int32) for ei in eis]
    ds = [ei[1].astype(jnp.int32) for ei in eis]
    tas = [x_atom, x_bond, x_motif]
    tbs = _build_relu_table(tas)

    zfeat = jnp.zeros((CHUNK, H), jnp.float32)
    ones = jnp.ones((CHUNK, H), jnp.float32)

    sums, cnt = _sc_aggregate(tas, tbs, ss, ds, zfeat, ones)

    cell_pad = jnp.pad(x_cell, ((0, NP - N), (0, 0)))
    out = _dense_chain(sums, cnt, cell_pad, Wl, bl, Wr,
                       Wproj, bproj.reshape(1, H), Wout, bout.reshape(1, 1))
    return out[:N]
